# baseline (speedup 1.0000x reference)
import functools

import jax
import jax.numpy as jnp
from jax import lax
from jax.experimental import pallas as pl
from jax.experimental.pallas import tpu as pltpu
from jax.experimental.pallas import tpu_sc as plsc

S = 20.0
M = 0.2
LOG2E = 1.4426950408889634
LN2 = 0.6931471805599453

CSPLIT = 98304
RT = 2048


def _main_body(inp_ref, lab_ref, out_ref, m_s, s_s, t_s, loss_s, *, C, Rb, Cb):
    i = pl.program_id(0)
    j = pl.program_id(1)
    nr = pl.num_programs(0)
    nc = pl.num_programs(1)
    K2 = S * LOG2E

    @pl.when(j == 0)
    def _():
        m_s[...] = jnp.full((Rb, 1), -jnp.inf, jnp.float32)
        s_s[...] = jnp.zeros((Rb, 1), jnp.float32)
        t_s[...] = jnp.zeros((Rb, 1), jnp.float32)

    @pl.when((i == 0) & (j == 0))
    def _():
        loss_s[0] = 0.0

    def tile(ragged):
        cos = inp_ref[...]
        lane = lax.broadcasted_iota(jnp.int32, (Rb, Cb), 1)
        islab = lane == (lab_ref[...] - j * Cb)
        t_s[...] += jnp.sum(jnp.where(islab, cos, 0.0), axis=1, keepdims=True)
        if ragged:
            rem = C - (C // Cb) * Cb
            cos = jnp.where(lane < rem, cos, -jnp.inf)
        mloc = K2 * jnp.max(cos, axis=1, keepdims=True)
        mold = m_s[...]
        mnew = jnp.maximum(mold, mloc)
        m_s[...] = mnew
        s_s[...] = s_s[...] * jnp.exp2(mold - mnew) + jnp.sum(
            jnp.exp2(K2 * cos - mnew), axis=1, keepdims=True
        )

    @pl.when(j < nc - 1)
    def _():
        tile(False)

    @pl.when(j == nc - 1)
    def _():
        tile(True)

    @pl.when(j == nc - 1)
    def _():
        m2 = m_s[...]
        t = t_s[...]
        mS = m2 * LN2
        a = jnp.exp(S * t - mS)
        b = jnp.exp(S * (t - M) - mS)
        sp = s_s[...] - a + b
        lse = mS + jnp.log(sp)
        loss_s[0] += jnp.sum(lse - S * (t - M))

    @pl.when((i == nr - 1) & (j == nc - 1))
    def _():
        out_ref[0] = loss_s[0]


def _make_sc_stream(C, Rt, nrows, Csc, num_cores, NS, Wc, G):
    NW = num_cores * NS
    rpw = nrows // NW
    n_chunks = Csc // Wc
    n_groups = Wc // (16 * G)
    mesh = plsc.VectorSubcoreMesh(
        core_axis_name="c", subcore_axis_name="s", num_cores=num_cores
    )
    NC = num_cores

    @functools.partial(
        pl.kernel,
        mesh=mesh,
        compiler_params=pltpu.CompilerParams(needs_layout_passes=False),
        out_type=[
            jax.ShapeDtypeStruct((nrows * 16,), jnp.float32),
            jax.ShapeDtypeStruct((nrows * 16,), jnp.float32),
            jax.ShapeDtypeStruct((nrows * 16,), jnp.float32),
        ],
        scratch_types=[
            pltpu.VMEM((Wc,), jnp.float32),
            pltpu.VMEM((Wc,), jnp.float32),
            pltpu.VMEM((rpw,), jnp.int32),
            pltpu.VMEM((rpw * 16,), jnp.float32),
            pltpu.VMEM((rpw * 16,), jnp.float32),
            pltpu.VMEM((rpw * 16,), jnp.float32),
            pltpu.SemaphoreType.DMA,
            pltpu.SemaphoreType.DMA,
        ],
    )
    def sc_stream(
        in_hbm, lab_hbm, m_hbm, s_hbm, t_hbm,
        buf0, buf1, lab_v, m_v, s_v, t_v, sem0, sem1,
    ):
        wid = lax.axis_index("s") * NC + lax.axis_index("c")
        r0 = wid * rpw
        pltpu.sync_copy(lab_hbm.at[pl.ds(Rt + r0, rpw)], lab_v)
        iota16 = lax.iota(jnp.int32, 16)
        bufs = (buf0, buf1)
        sems = (sem0, sem1)

        def row_body(r, _):
            row = Rt + r0 + r
            lab16 = plsc.load_gather(lab_v, [jnp.full((16,), r, jnp.int32)])

            m16 = jnp.full((16,), -jnp.inf, jnp.float32)
            s16 = jnp.zeros((16,), jnp.float32)
            t16 = jnp.zeros((16,), jnp.float32)

            cps = [None] * n_chunks
            cps[0] = pltpu.async_copy(in_hbm.at[row, pl.ds(0, Wc)], buf0, sem0)
            for ci in range(n_chunks):
                b = ci % 2
                buf = bufs[b]
                if ci + 1 < n_chunks:
                    cps[ci + 1] = pltpu.async_copy(
                        in_hbm.at[row, pl.ds((ci + 1) * Wc, Wc)],
                        bufs[1 - b],
                        sems[1 - b],
                    )
                cps[ci].wait()
                c0 = ci * Wc

                def group_body(gi, carry2):
                    gm16, gs16 = carry2
                    base = gi * (16 * G)
                    ys = [S * buf[pl.ds(base + k * 16, 16)] for k in range(G)]
                    mg = ys[0]
                    for y in ys[1:]:
                        mg = jnp.maximum(mg, y)
                    mnew = jnp.maximum(gm16, mg)
                    acc = gs16 * jnp.exp(gm16 - mnew)
                    for y in ys:
                        acc = acc + jnp.exp(y - mnew)
                    return mnew, acc

                m16, s16 = lax.fori_loop(
                    0, n_groups, group_body, (m16, s16), unroll=2
                )
                off16 = lab16 - c0
                idx16 = jnp.minimum(jnp.maximum(off16, 0), Wc - 1)
                g16 = plsc.load_gather(buf, [idx16])
                hit = (off16 >= 0) & (off16 < Wc) & (iota16 == 0)
                t16 = t16 + jnp.where(hit, g16, 0.0)

            m_v[pl.ds(r * 16, 16)] = m16
            s_v[pl.ds(r * 16, 16)] = s16
            t_v[pl.ds(r * 16, 16)] = t16
            return 0

        lax.fori_loop(0, rpw, row_body, 0)
        pltpu.sync_copy(m_v, m_hbm.at[pl.ds(r0 * 16, rpw * 16)])
        pltpu.sync_copy(s_v, s_hbm.at[pl.ds(r0 * 16, rpw * 16)])
        pltpu.sync_copy(t_v, t_hbm.at[pl.ds(r0 * 16, rpw * 16)])

    return sc_stream


def _tail_body(inp_ref, lab_ref, mo_ref, so_ref, to_ref, *, C, Rb, Cb):
    K2 = S * LOG2E
    cos = inp_ref[...]
    lane = lax.broadcasted_iota(jnp.int32, (Rb, Cb), 1)
    islab = lane == (lab_ref[...] - CSPLIT)
    to_ref[...] = jnp.sum(jnp.where(islab, cos, 0.0), axis=1, keepdims=True)
    rem = C - CSPLIT
    cosm = jnp.where(lane < rem, cos, -jnp.inf)
    m2 = K2 * jnp.max(cosm, axis=1, keepdims=True)
    mo_ref[...] = m2
    so_ref[...] = jnp.sum(jnp.exp2(K2 * cosm - m2), axis=1, keepdims=True)


def _combine_body(
    pa_ref, ma_ref, sa_ref, ta_ref, mb_ref, sb_ref, tb_ref, out_ref, *, B
):
    m16 = ma_ref[...]
    s16 = sa_ref[...]
    m_b = mb_ref[...] * LN2
    m = jnp.maximum(jnp.max(m16, axis=1, keepdims=True), m_b)
    s = jnp.sum(s16 * jnp.exp(m16 - m), axis=1, keepdims=True) + sb_ref[
        ...
    ] * jnp.exp(m_b - m)
    t = jnp.sum(ta_ref[...], axis=1, keepdims=True) + tb_ref[...]
    a = jnp.exp(S * t - m)
    b = jnp.exp(S * (t - M) - m)
    sp = s - a + b
    lse = m + jnp.log(sp)
    out_ref[0] = (pa_ref[0] + jnp.sum(lse - S * (t - M))) / B


@jax.jit
def kernel(input, labels):
    B, C = input.shape
    lab1 = labels.reshape(B).astype(jnp.int32)
    lab2 = lab1.reshape(B, 1)
    Bs = B - RT

    info = plsc.get_sparse_core_info()
    half = Bs // 2
    sc_stream0 = _make_sc_stream(
        C, RT, half, CSPLIT, 1, info.num_subcores, Wc=8192, G=16
    )
    sc_stream1 = _make_sc_stream(
        C, RT + half, half, CSPLIT, 1, info.num_subcores, Wc=8192, G=16
    )
    m0, s0, t0 = sc_stream0(input, lab1)
    m1, s1, t1 = sc_stream1(input, lab1)
    m_a = jnp.concatenate([m0, m1])
    s_a = jnp.concatenate([s0, s1])
    t_a = jnp.concatenate([t0, t1])

    Rb, Cb = 512, 4096
    nr = RT // Rb
    nc = pl.cdiv(C, Cb)
    pa = pl.pallas_call(
        functools.partial(_main_body, C=C, Rb=Rb, Cb=Cb),
        grid=(nr, nc),
        in_specs=[
            pl.BlockSpec((Rb, Cb), lambda i, j: (i, j)),
            pl.BlockSpec((Rb, 1), lambda i, j: (i, 0)),
        ],
        out_specs=pl.BlockSpec(memory_space=pltpu.SMEM),
        out_shape=jax.ShapeDtypeStruct((1,), jnp.float32),
        scratch_shapes=[
            pltpu.VMEM((Rb, 1), jnp.float32),
            pltpu.VMEM((Rb, 1), jnp.float32),
            pltpu.VMEM((Rb, 1), jnp.float32),
            pltpu.SMEM((1,), jnp.float32),
        ],
    )(input, lab2)

    Rb2 = 512
    Cb2 = 2048
    nrt = Bs // Rb2
    cblk = CSPLIT // Cb2
    rblk0 = RT // Rb2
    m_b, s_b, t_b = pl.pallas_call(
        functools.partial(_tail_body, C=C, Rb=Rb2, Cb=Cb2),
        grid=(nrt,),
        in_specs=[
            pl.BlockSpec((Rb2, Cb2), lambda i: (rblk0 + i, cblk)),
            pl.BlockSpec((Rb2, 1), lambda i: (rblk0 + i, 0)),
        ],
        out_specs=[
            pl.BlockSpec((Rb2, 1), lambda i: (i, 0)),
            pl.BlockSpec((Rb2, 1), lambda i: (i, 0)),
            pl.BlockSpec((Rb2, 1), lambda i: (i, 0)),
        ],
        out_shape=[
            jax.ShapeDtypeStruct((Bs, 1), jnp.float32),
            jax.ShapeDtypeStruct((Bs, 1), jnp.float32),
            jax.ShapeDtypeStruct((Bs, 1), jnp.float32),
        ],
    )(input, lab2)

    out = pl.pallas_call(
        functools.partial(_combine_body, B=B),
        in_specs=[
            pl.BlockSpec(memory_space=pltpu.SMEM),
            pl.BlockSpec((Bs, 16), lambda: (0, 0)),
            pl.BlockSpec((Bs, 16), lambda: (0, 0)),
            pl.BlockSpec((Bs, 16), lambda: (0, 0)),
            pl.BlockSpec((Bs, 1), lambda: (0, 0)),
            pl.BlockSpec((Bs, 1), lambda: (0, 0)),
            pl.BlockSpec((Bs, 1), lambda: (0, 0)),
        ],
        out_specs=pl.BlockSpec(memory_space=pltpu.SMEM),
        out_shape=jax.ShapeDtypeStruct((1,), jnp.float32),
    )(
        pa,
        m_a.reshape(Bs, 16),
        s_a.reshape(Bs, 16),
        t_a.reshape(Bs, 16),
        m_b,
        s_b,
        t_b,
    )
    return out[0]


# speedup vs baseline: 1.4807x; 1.4807x over previous
import functools

import jax
import jax.numpy as jnp
from jax import lax
from jax.experimental import pallas as pl
from jax.experimental.pallas import tpu as pltpu
from jax.experimental.pallas import tpu_sc as plsc

S = 20.0
M = 0.2
LOG2E = 1.4426950408889634
LN2 = 0.6931471805599453

CSPLIT = 98304
RT = 2816


def _main_body(inp_ref, lab_ref, out_ref, m_s, s_s, t_s, loss_s, *, C, Rb, Cb):
    i = pl.program_id(0)
    j = pl.program_id(1)
    nr = pl.num_programs(0)
    nc = pl.num_programs(1)
    K2 = S * LOG2E

    @pl.when(j == 0)
    def _():
        m_s[...] = jnp.full((Rb, 1), -jnp.inf, jnp.float32)
        s_s[...] = jnp.zeros((Rb, 1), jnp.float32)
        t_s[...] = jnp.zeros((Rb, 1), jnp.float32)

    @pl.when((i == 0) & (j == 0))
    def _():
        loss_s[0] = 0.0

    def tile(ragged):
        cos = inp_ref[...]
        lane = lax.broadcasted_iota(jnp.int32, (Rb, Cb), 1)
        islab = lane == (lab_ref[...] - j * Cb)
        t_s[...] += jnp.sum(jnp.where(islab, cos, 0.0), axis=1, keepdims=True)
        if ragged:
            rem = C - (C // Cb) * Cb
            cos = jnp.where(lane < rem, cos, -jnp.inf)
        mloc = K2 * jnp.max(cos, axis=1, keepdims=True)
        mold = m_s[...]
        mnew = jnp.maximum(mold, mloc)
        m_s[...] = mnew
        s_s[...] = s_s[...] * jnp.exp2(mold - mnew) + jnp.sum(
            jnp.exp2(K2 * cos - mnew), axis=1, keepdims=True
        )

    @pl.when(j < nc - 1)
    def _():
        tile(False)

    @pl.when(j == nc - 1)
    def _():
        tile(True)

    @pl.when(j == nc - 1)
    def _():
        m2 = m_s[...]
        t = t_s[...]
        mS = m2 * LN2
        a = jnp.exp(S * t - mS)
        b = jnp.exp(S * (t - M) - mS)
        sp = s_s[...] - a + b
        lse = mS + jnp.log(sp)
        loss_s[0] += jnp.sum(lse - S * (t - M))

    @pl.when((i == nr - 1) & (j == nc - 1))
    def _():
        out_ref[0] = loss_s[0]


def _make_sc_stream(C, Rt, nrows, Csc, num_cores, NS, Wc, G):
    NW = num_cores * NS
    rpw = nrows // NW
    n_chunks = Csc // Wc
    n_groups = Wc // (16 * G)
    mesh = plsc.VectorSubcoreMesh(
        core_axis_name="c", subcore_axis_name="s", num_cores=num_cores
    )
    NC = num_cores

    @functools.partial(
        pl.kernel,
        mesh=mesh,
        compiler_params=pltpu.CompilerParams(needs_layout_passes=False),
        out_type=[
            jax.ShapeDtypeStruct((nrows * 16,), jnp.float32),
            jax.ShapeDtypeStruct((nrows * 16,), jnp.float32),
            jax.ShapeDtypeStruct((nrows * 16,), jnp.float32),
        ],
        scratch_types=[
            pltpu.VMEM((Wc,), jnp.float32),
            pltpu.VMEM((Wc,), jnp.float32),
            pltpu.VMEM((rpw,), jnp.int32),
            pltpu.VMEM((rpw * 16,), jnp.float32),
            pltpu.VMEM((rpw * 16,), jnp.float32),
            pltpu.VMEM((rpw * 16,), jnp.float32),
            pltpu.SemaphoreType.DMA,
            pltpu.SemaphoreType.DMA,
        ],
    )
    def sc_stream(
        in_hbm, lab_hbm, m_hbm, s_hbm, t_hbm,
        buf0, buf1, lab_v, m_v, s_v, t_v, sem0, sem1,
    ):
        wid = lax.axis_index("s") * NC + lax.axis_index("c")
        r0 = wid * rpw
        pltpu.sync_copy(lab_hbm.at[pl.ds(Rt + r0, rpw)], lab_v)
        iota16 = lax.iota(jnp.int32, 16)
        bufs = (buf0, buf1)
        sems = (sem0, sem1)

        def row_body(r, _):
            row = Rt + r0 + r
            lab16 = plsc.load_gather(lab_v, [jnp.full((16,), r, jnp.int32)])

            m16 = jnp.full((16,), -jnp.inf, jnp.float32)
            s16 = jnp.zeros((16,), jnp.float32)
            t16 = jnp.zeros((16,), jnp.float32)

            cps = [None] * n_chunks
            cps[0] = pltpu.async_copy(in_hbm.at[row, pl.ds(0, Wc)], buf0, sem0)
            for ci in range(n_chunks):
                b = ci % 2
                buf = bufs[b]
                if ci + 1 < n_chunks:
                    cps[ci + 1] = pltpu.async_copy(
                        in_hbm.at[row, pl.ds((ci + 1) * Wc, Wc)],
                        bufs[1 - b],
                        sems[1 - b],
                    )
                cps[ci].wait()
                c0 = ci * Wc

                def group_body(gi, carry2):
                    gm16, gs16 = carry2
                    base = gi * (16 * G)
                    ys = [S * buf[pl.ds(base + k * 16, 16)] for k in range(G)]
                    mg = ys[0]
                    for y in ys[1:]:
                        mg = jnp.maximum(mg, y)
                    mnew = jnp.maximum(gm16, mg)
                    acc = gs16 * jnp.exp(gm16 - mnew)
                    for y in ys:
                        acc = acc + jnp.exp(y - mnew)
                    return mnew, acc

                m16, s16 = lax.fori_loop(
                    0, n_groups, group_body, (m16, s16), unroll=2
                )
                off16 = lab16 - c0
                idx16 = jnp.minimum(jnp.maximum(off16, 0), Wc - 1)
                g16 = plsc.load_gather(buf, [idx16])
                hit = (off16 >= 0) & (off16 < Wc) & (iota16 == 0)
                t16 = t16 + jnp.where(hit, g16, 0.0)

            m_v[pl.ds(r * 16, 16)] = m16
            s_v[pl.ds(r * 16, 16)] = s16
            t_v[pl.ds(r * 16, 16)] = t16
            return 0

        lax.fori_loop(0, rpw, row_body, 0)
        pltpu.sync_copy(m_v, m_hbm.at[pl.ds(r0 * 16, rpw * 16)])
        pltpu.sync_copy(s_v, s_hbm.at[pl.ds(r0 * 16, rpw * 16)])
        pltpu.sync_copy(t_v, t_hbm.at[pl.ds(r0 * 16, rpw * 16)])

    return sc_stream


def _tail_body(inp_ref, lab_ref, mo_ref, so_ref, to_ref, *, C, Rb, Cb):
    K2 = S * LOG2E
    cos = inp_ref[...]
    lane = lax.broadcasted_iota(jnp.int32, (Rb, Cb), 1)
    islab = lane == (lab_ref[...] - CSPLIT)
    to_ref[...] = jnp.sum(jnp.where(islab, cos, 0.0), axis=1, keepdims=True)
    rem = C - CSPLIT
    cosm = jnp.where(lane < rem, cos, -jnp.inf)
    m2 = K2 * jnp.max(cosm, axis=1, keepdims=True)
    mo_ref[...] = m2
    so_ref[...] = jnp.sum(jnp.exp2(K2 * cosm - m2), axis=1, keepdims=True)


def _combine_body(
    pa_ref, ma_ref, sa_ref, ta_ref, mb_ref, sb_ref, tb_ref, out_ref, *, B
):
    m16 = ma_ref[...]
    s16 = sa_ref[...]
    m_b = mb_ref[...] * LN2
    m = jnp.maximum(jnp.max(m16, axis=1, keepdims=True), m_b)
    s = jnp.sum(s16 * jnp.exp(m16 - m), axis=1, keepdims=True) + sb_ref[
        ...
    ] * jnp.exp(m_b - m)
    t = jnp.sum(ta_ref[...], axis=1, keepdims=True) + tb_ref[...]
    a = jnp.exp(S * t - m)
    b = jnp.exp(S * (t - M) - m)
    sp = s - a + b
    lse = m + jnp.log(sp)
    out_ref[0] = (pa_ref[0] + jnp.sum(lse - S * (t - M))) / B


@jax.jit
def kernel(input, labels):
    B, C = input.shape
    lab1 = labels.reshape(B).astype(jnp.int32)
    lab2 = lab1.reshape(B, 1)
    Bs = B - RT

    info = plsc.get_sparse_core_info()
    sc_stream = _make_sc_stream(
        C, RT, Bs, CSPLIT, info.num_cores, info.num_subcores, Wc=8192, G=16
    )
    m_a, s_a, t_a = sc_stream(input, lab1)

    Rb, Cb = 256, 4096
    nr = RT // Rb
    nc = pl.cdiv(C, Cb)
    pa = pl.pallas_call(
        functools.partial(_main_body, C=C, Rb=Rb, Cb=Cb),
        grid=(nr, nc),
        in_specs=[
            pl.BlockSpec((Rb, Cb), lambda i, j: (i, j)),
            pl.BlockSpec((Rb, 1), lambda i, j: (i, 0)),
        ],
        out_specs=pl.BlockSpec(memory_space=pltpu.SMEM),
        out_shape=jax.ShapeDtypeStruct((1,), jnp.float32),
        scratch_shapes=[
            pltpu.VMEM((Rb, 1), jnp.float32),
            pltpu.VMEM((Rb, 1), jnp.float32),
            pltpu.VMEM((Rb, 1), jnp.float32),
            pltpu.SMEM((1,), jnp.float32),
        ],
    )(input, lab2)

    Rb2 = 256
    Cb2 = 2048
    nrt = Bs // Rb2
    cblk = CSPLIT // Cb2
    rblk0 = RT // Rb2
    m_b, s_b, t_b = pl.pallas_call(
        functools.partial(_tail_body, C=C, Rb=Rb2, Cb=Cb2),
        grid=(nrt,),
        in_specs=[
            pl.BlockSpec((Rb2, Cb2), lambda i: (rblk0 + i, cblk)),
            pl.BlockSpec((Rb2, 1), lambda i: (rblk0 + i, 0)),
        ],
        out_specs=[
            pl.BlockSpec((Rb2, 1), lambda i: (i, 0)),
            pl.BlockSpec((Rb2, 1), lambda i: (i, 0)),
            pl.BlockSpec((Rb2, 1), lambda i: (i, 0)),
        ],
        out_shape=[
            jax.ShapeDtypeStruct((Bs, 1), jnp.float32),
            jax.ShapeDtypeStruct((Bs, 1), jnp.float32),
            jax.ShapeDtypeStruct((Bs, 1), jnp.float32),
        ],
    )(input, lab2)

    out = pl.pallas_call(
        functools.partial(_combine_body, B=B),
        in_specs=[
            pl.BlockSpec(memory_space=pltpu.SMEM),
            pl.BlockSpec((Bs, 16), lambda: (0, 0)),
            pl.BlockSpec((Bs, 16), lambda: (0, 0)),
            pl.BlockSpec((Bs, 16), lambda: (0, 0)),
            pl.BlockSpec((Bs, 1), lambda: (0, 0)),
            pl.BlockSpec((Bs, 1), lambda: (0, 0)),
            pl.BlockSpec((Bs, 1), lambda: (0, 0)),
        ],
        out_specs=pl.BlockSpec(memory_space=pltpu.SMEM),
        out_shape=jax.ShapeDtypeStruct((1,), jnp.float32),
    )(
        pa,
        m_a.reshape(Bs, 16),
        s_a.reshape(Bs, 16),
        t_a.reshape(Bs, 16),
        m_b,
        s_b,
        t_b,
    )
    return out[0]


# speedup vs baseline: 5.3626x; 3.6218x over previous
import functools

import jax
import jax.numpy as jnp
from jax import lax
from jax.experimental import pallas as pl
from jax.experimental.pallas import tpu as pltpu

S = 20.0
M = 0.2
LOG2E = 1.4426950408889634
LN2 = 0.6931471805599453


def _body(inp_ref, lab_ref, out_ref, m_s, s_s, t_s, *, C, B, Cb):
    j = pl.program_id(0)
    nc = pl.num_programs(0)
    K2 = S * LOG2E

    @pl.when(j == 0)
    def _():
        m_s[...] = jnp.full((1, B), -jnp.inf, jnp.float32)
        s_s[...] = jnp.zeros((1, B), jnp.float32)
        t_s[...] = jnp.zeros((1, B), jnp.float32)

    def tile(ragged):
        cos = inp_ref[...]
        cls = lax.broadcasted_iota(jnp.int32, (Cb, B), 0)
        islab = cls == (lab_ref[...] - j * Cb)
        t_s[...] += jnp.sum(jnp.where(islab, cos, 0.0), axis=0, keepdims=True)
        if ragged:
            rem = C - (C // Cb) * Cb
            cos = jnp.where(cls < rem, cos, -jnp.inf)
        mloc = K2 * jnp.max(cos, axis=0, keepdims=True)
        mold = m_s[...]
        mnew = jnp.maximum(mold, mloc)
        m_s[...] = mnew
        s_s[...] = s_s[...] * jnp.exp2(mold - mnew) + jnp.sum(
            jnp.exp2(K2 * cos - mnew), axis=0, keepdims=True
        )

    @pl.when(j < nc - 1)
    def _():
        tile(False)

    @pl.when(j == nc - 1)
    def _():
        tile(True)

    @pl.when(j == nc - 1)
    def _():
        m2 = m_s[...]
        t = t_s[...]
        mS = m2 * LN2
        a = jnp.exp(S * t - mS)
        b = jnp.exp(S * (t - M) - mS)
        sp = s_s[...] - a + b
        lse = mS + jnp.log(sp)
        out_ref[0] = jnp.sum(lse - S * (t - M)) / B


@jax.jit
def kernel(input, labels):
    B, C = input.shape
    lab = labels.reshape(1, B).astype(jnp.int32)
    inpT = input.T
    Cb = 1024
    nc = pl.cdiv(C, Cb)
    out = pl.pallas_call(
        functools.partial(_body, C=C, B=B, Cb=Cb),
        grid=(nc,),
        in_specs=[
            pl.BlockSpec((Cb, B), lambda j: (j, 0)),
            pl.BlockSpec((1, B), lambda j: (0, 0)),
        ],
        out_specs=pl.BlockSpec(memory_space=pltpu.SMEM),
        out_shape=jax.ShapeDtypeStruct((1,), jnp.float32),
        scratch_shapes=[
            pltpu.VMEM((1, B), jnp.float32),
            pltpu.VMEM((1, B), jnp.float32),
            pltpu.VMEM((1, B), jnp.float32),
        ],
    )(inpT, lab)
    return out[0]
